# baseline (speedup 1.0000x reference)
import functools

import jax
import jax.numpy as jnp
from jax import lax
from jax.experimental import pallas as pl
from jax.experimental.pallas import tpu as pltpu
from jax.experimental.pallas import tpu_sc as plsc

NUM_CORES = 2
NUM_SUBCORES = 16
LANES = 16
NUM_WORKERS = NUM_CORES * NUM_SUBCORES
CHUNK = 64


@functools.cache
def _make_lookup_kernel(batch, words):
    assert batch % (8 * NUM_WORKERS) == 0
    bpw = batch // NUM_WORKERS
    nchunks = bpw // CHUNK
    groups = CHUNK // LANES
    mesh = plsc.VectorSubcoreMesh(core_axis_name="c", subcore_axis_name="s")

    @functools.partial(
        pl.kernel,
        out_type=jax.ShapeDtypeStruct((batch,), jnp.float32),
        mesh=mesh,
        compiler_params=pltpu.CompilerParams(
            needs_layout_passes=False, use_tc_tiling_on_sc=True
        ),
        scratch_types=[
            pltpu.VMEM((bpw,), jnp.int32),
            pltpu.VMEM((bpw,), jnp.int32),
            pltpu.VMEM((CHUNK, words), jnp.float32),
            pltpu.VMEM((CHUNK, words), jnp.float32),
            pltpu.VMEM((CHUNK, words), jnp.float32),
            pltpu.VMEM((CHUNK, words), jnp.float32),
            pltpu.VMEM((bpw,), jnp.float32),
            pltpu.SemaphoreType.DMA,
            pltpu.SemaphoreType.DMA,
            pltpu.SemaphoreType.DMA,
            pltpu.SemaphoreType.DMA,
            pltpu.SemaphoreType.DMA,
        ],
    )
    def sc_kernel(movies_hbm, users_hbm, mtab_hbm, utab_hbm, out_hbm,
                  midx_v, uidx_v, em0, em1, eu0, eu1, outv,
                  sem_i, sm0, sm1, su0, su1):
        wid = lax.axis_index("s") * NUM_CORES + lax.axis_index("c")
        base = wid * bpw

        cim = pltpu.async_copy(movies_hbm.at[pl.ds(base, bpw)], midx_v, sem_i)
        ciu = pltpu.async_copy(users_hbm.at[pl.ds(base, bpw)], uidx_v, sem_i)
        cim.wait()
        ciu.wait()

        zeros = jnp.zeros((LANES,), jnp.float32)
        embuf = (em0, em1)
        eubuf = (eu0, eu1)
        smbuf = (sm0, sm1)
        subuf = (su0, su1)

        def fetch(c, em_v, eu_v, sm, su):
            b0 = c * CHUNK

            def fbody(g, _):
                mrow = midx_v[pl.ds(b0 + g * LANES, LANES)]
                urow = uidx_v[pl.ds(b0 + g * LANES, LANES)]
                for j in range(LANES):
                    pltpu.async_copy(
                        mtab_hbm.at[pl.ds(mrow[j], 1), :],
                        em_v.at[pl.ds(g * LANES + j, 1), :], sm,
                    )
                    pltpu.async_copy(
                        utab_hbm.at[pl.ds(urow[j], 1), :],
                        eu_v.at[pl.ds(g * LANES + j, 1), :], su,
                    )
                return 0

            lax.fori_loop(0, groups, fbody, 0)

        def drain(em_v, eu_v, sm, su):
            pltpu.make_async_copy(
                mtab_hbm.at[pl.ds(0, CHUNK), :], em_v, sm
            ).wait()
            pltpu.make_async_copy(
                utab_hbm.at[pl.ds(0, CHUNK), :], eu_v, su
            ).wait()

        def compute(c, em_v, eu_v):
            b0 = c * CHUNK

            def cbody(g, _):
                acc_slot = b0 + g * LANES
                outv[pl.ds(acc_slot, LANES)] = zeros
                for j in range(LANES):
                    k = g * LANES + j
                    mw = em_v[k, pl.ds(0, words)]
                    uw = eu_v[k, pl.ds(0, words)]
                    ma, mb = plsc.unpack(
                        plsc.bitcast(mw, jnp.bfloat16),
                        format=plsc.PackFormat.INTERLEAVED,
                        preferred_element_type=jnp.float32,
                    )
                    ua, ub = plsc.unpack(
                        plsc.bitcast(uw, jnp.bfloat16),
                        format=plsc.PackFormat.INTERLEAVED,
                        preferred_element_type=jnp.float32,
                    )
                    part = ma * ua + mb * ub
                    plsc.addupdate_scatter(
                        outv,
                        [jnp.zeros((LANES,), jnp.int32) + (acc_slot + j)],
                        part,
                    )
                return 0

            lax.fori_loop(0, groups, cbody, 0)

        fetch(0, embuf[0], eubuf[0], smbuf[0], subuf[0])
        for c in range(nchunks):
            p = c % 2
            if c + 1 < nchunks:
                q = (c + 1) % 2
                fetch(c + 1, embuf[q], eubuf[q], smbuf[q], subuf[q])
            drain(embuf[p], eubuf[p], smbuf[p], subuf[p])
            compute(c, embuf[p], eubuf[p])

        pltpu.sync_copy(outv, out_hbm.at[pl.ds(base, bpw)])

    return jax.jit(sc_kernel)


def _pack_table(table):
    rows, dim = table.shape
    t16 = table.astype(jnp.bfloat16).reshape(rows, dim // 2, 2)
    return lax.bitcast_convert_type(t16, jnp.float32)


def kernel(movies, users, movie_table, user_table):
    batch = movies.shape[0]
    words = movie_table.shape[1] // 2
    out = _make_lookup_kernel(batch, words)(
        movies.astype(jnp.int32), users.astype(jnp.int32),
        _pack_table(movie_table), _pack_table(user_table)
    )
    return out.reshape(batch, 1)


# speedup vs baseline: 3.0229x; 3.0229x over previous
import functools

import jax
import jax.numpy as jnp
from jax import lax
from jax.experimental import pallas as pl
from jax.experimental.pallas import tpu as pltpu
from jax.experimental.pallas import tpu_sc as plsc

NUM_CORES = 2
NUM_SUBCORES = 16
LANES = 16
NUM_WORKERS = NUM_CORES * NUM_SUBCORES
CHUNK = 64


@functools.cache
def _make_lookup_kernel(batch, dim):
    assert batch % (8 * NUM_WORKERS) == 0
    bpw = batch // NUM_WORKERS
    nchunks = bpw // CHUNK
    groups = CHUNK // LANES
    mesh = plsc.VectorSubcoreMesh(core_axis_name="c", subcore_axis_name="s")

    @functools.partial(
        pl.kernel,
        out_type=jax.ShapeDtypeStruct((batch,), jnp.float32),
        mesh=mesh,
        compiler_params=pltpu.CompilerParams(
            needs_layout_passes=False, use_tc_tiling_on_sc=True
        ),
        scratch_types=[
            pltpu.VMEM((bpw,), jnp.int32),
            pltpu.VMEM((bpw,), jnp.int32),
            pltpu.VMEM((CHUNK, dim), jnp.float32),
            pltpu.VMEM((CHUNK, dim), jnp.float32),
            pltpu.VMEM((CHUNK, dim), jnp.float32),
            pltpu.VMEM((CHUNK, dim), jnp.float32),
            pltpu.VMEM((bpw,), jnp.float32),
            pltpu.SemaphoreType.DMA,
            pltpu.SemaphoreType.DMA,
            pltpu.SemaphoreType.DMA,
            pltpu.SemaphoreType.DMA,
            pltpu.SemaphoreType.DMA,
        ],
    )
    def sc_kernel(movies_hbm, users_hbm, mtab_hbm, utab_hbm, out_hbm,
                  midx_v, uidx_v, em0, em1, eu0, eu1, outv,
                  sem_i, sm0, sm1, su0, su1):
        wid = lax.axis_index("s") * NUM_CORES + lax.axis_index("c")
        base = wid * bpw

        cim = pltpu.async_copy(movies_hbm.at[pl.ds(base, bpw)], midx_v, sem_i)
        ciu = pltpu.async_copy(users_hbm.at[pl.ds(base, bpw)], uidx_v, sem_i)
        cim.wait()
        ciu.wait()

        half = dim // 2
        zeros = jnp.zeros((LANES,), jnp.float32)
        embuf = (em0, em1)
        eubuf = (eu0, eu1)
        smbuf = (sm0, sm1)
        subuf = (su0, su1)

        def fetch(c, em_v, eu_v, sm, su):
            b0 = c * CHUNK

            def fbody(g, _):
                mrow = midx_v[pl.ds(b0 + g * LANES, LANES)]
                urow = uidx_v[pl.ds(b0 + g * LANES, LANES)]
                for j in range(LANES):
                    pltpu.async_copy(
                        mtab_hbm.at[pl.ds(mrow[j], 1), :],
                        em_v.at[pl.ds(g * LANES + j, 1), :], sm,
                    )
                    pltpu.async_copy(
                        utab_hbm.at[pl.ds(urow[j], 1), :],
                        eu_v.at[pl.ds(g * LANES + j, 1), :], su,
                    )
                return 0

            lax.fori_loop(0, groups, fbody, 0)

        def drain(em_v, eu_v, sm, su):
            pltpu.make_async_copy(
                mtab_hbm.at[pl.ds(0, CHUNK), :], em_v, sm
            ).wait()
            pltpu.make_async_copy(
                utab_hbm.at[pl.ds(0, CHUNK), :], eu_v, su
            ).wait()

        def compute(c, em_v, eu_v):
            b0 = c * CHUNK

            def cbody(g, _):
                acc_slot = b0 + g * LANES
                outv[pl.ds(acc_slot, LANES)] = zeros
                for j in range(LANES):
                    k = g * LANES + j
                    em_lo = em_v[k, pl.ds(0, half)]
                    em_hi = em_v[k, pl.ds(half, half)]
                    eu_lo = eu_v[k, pl.ds(0, half)]
                    eu_hi = eu_v[k, pl.ds(half, half)]
                    part = em_lo * eu_lo + em_hi * eu_hi
                    plsc.addupdate_scatter(
                        outv,
                        [jnp.zeros((LANES,), jnp.int32) + (acc_slot + j)],
                        part,
                    )
                return 0

            lax.fori_loop(0, groups, cbody, 0)

        fetch(0, embuf[0], eubuf[0], smbuf[0], subuf[0])
        for c in range(nchunks):
            p = c % 2
            if c + 1 < nchunks:
                q = (c + 1) % 2
                fetch(c + 1, embuf[q], eubuf[q], smbuf[q], subuf[q])
            drain(embuf[p], eubuf[p], smbuf[p], subuf[p])
            compute(c, embuf[p], eubuf[p])

        pltpu.sync_copy(outv, out_hbm.at[pl.ds(base, bpw)])

    return jax.jit(sc_kernel)


def kernel(movies, users, movie_table, user_table):
    batch = movies.shape[0]
    dim = movie_table.shape[1]
    out = _make_lookup_kernel(batch, dim)(
        movies.astype(jnp.int32), users.astype(jnp.int32),
        movie_table, user_table
    )
    return out.reshape(batch, 1)
